# baseline (speedup 1.0000x reference)
import functools

import jax
import jax.numpy as jnp
from jax import lax
from jax.experimental import pallas as pl
from jax.experimental.pallas import tpu as pltpu
from jax.experimental.pallas import tpu_sc as plsc

_SKY_ID = 142
_LOSS_WEIGHT = 0.1
_REGRESS_VALUE = 1.8
_EPS = 1e-06

_B, _H, _W = 4, 512, 512
_HW = _H * _W

_NC, _NS, _L = 2, 16, 16
_NW = _NC * _NS
_B_SC = 2
_ROWS_W = _B_SC * _H // _NW
_ROWS = 16
_NCHUNK = _ROWS_W // _ROWS
_UNROLL = 2
_GCOLS = _UNROLL * _L

_R_TC = 256

_mesh = plsc.VectorSubcoreMesh(core_axis_name="c", subcore_axis_name="s")


def _rsqrt(nsq):
    seed = jnp.int32(0x5F3759DF) - (lax.bitcast_convert_type(nsq, jnp.int32) >> 1)
    r = lax.bitcast_convert_type(seed, jnp.float32)
    h = 0.5 * nsq
    r = r * (1.5 - h * r * r)
    r = r * (1.5 - h * r * r)
    return r


@functools.partial(
    pl.kernel,
    mesh=_mesh,
    out_type=jax.ShapeDtypeStruct((_NW, 3 * _L), jnp.float32),
    scratch_types=[
        pltpu.VMEM((_ROWS, _W), jnp.float32),
        pltpu.VMEM((_ROWS, _W), jnp.float32),
        pltpu.VMEM((_ROWS, _W), jnp.int32),
        pltpu.VMEM((_ROWS, _W), jnp.int32),
        pltpu.VMEM((_ROWS, _W), jnp.float32),
        pltpu.VMEM((_ROWS, _W), jnp.float32),
        pltpu.VMEM((_ROWS, _W), jnp.float32),
        pltpu.VMEM((_ROWS, _W), jnp.float32),
        pltpu.VMEM((_ROWS, _W), jnp.float32),
        pltpu.VMEM((_ROWS, _W), jnp.float32),
        pltpu.VMEM((3 * _L,), jnp.float32),
        pltpu.SemaphoreType.DMA,
        pltpu.SemaphoreType.DMA,
    ],
)
def _sky_sc(pred_hbm, sem_hbm, nrm_hbm, out_hbm,
            pb0, pb1, sb0, sb1, xb0, xb1, yb0, yb1, zb0, zb1,
            accb, dsem0, dsem1):
    wid = lax.axis_index("c") * _NS + lax.axis_index("s")
    b = wid >> 4
    row0 = (wid & 15) * _ROWS_W

    bufs = ((pb0, sb0, xb0, yb0, zb0, dsem0),
            (pb1, sb1, xb1, yb1, zb1, dsem1))

    def issue(ci, pb, sb, xb, yb, zb, dsem):
        rows = pl.ds(row0 + ci * _ROWS, _ROWS)
        return [
            pltpu.async_copy(pred_hbm.at[b, rows, :], pb, dsem),
            pltpu.async_copy(sem_hbm.at[b, rows, :], sb, dsem),
            pltpu.async_copy(nrm_hbm.at[b, 0, rows, :], xb, dsem),
            pltpu.async_copy(nrm_hbm.at[b, 1, rows, :], yb, dsem),
            pltpu.async_copy(nrm_hbm.at[b, 2, rows, :], zb, dsem),
        ]

    def compute_chunk(pb, sb, xb, yb, zb, carry):
        def body(g, carry):
            r = g >> 4
            c0 = (g & 15) * _GCOLS
            new = list(carry)
            for u in range(_UNROLL):
                cols = pl.ds(c0 + u * _L, _L)
                cnt, l1, al = new[3 * u], new[3 * u + 1], new[3 * u + 2]
                sky = sb[r, cols] == _SKY_ID
                p = pb[r, cols]
                x = xb[r, cols]
                y = yb[r, cols]
                z = zb[r, cols]
                cnt = cnt + jnp.where(sky, 1.0, 0.0).astype(jnp.float32)
                l1 = l1 + jnp.where(sky, jnp.abs(p - _REGRESS_VALUE), 0.0)
                nsq = jnp.maximum(x * x + y * y + z * z, 1e-16)
                e = y * _rsqrt(nsq)
                valid = sky & (e > -0.999) & (e < 0.999)
                al = al + jnp.where(valid, 1.0 + e, 0.0)
                new[3 * u], new[3 * u + 1], new[3 * u + 2] = cnt, l1, al
            return tuple(new)

        return lax.fori_loop(0, _ROWS * (_W // _GCOLS), body, carry)

    zero = jnp.zeros((_L,), jnp.float32)
    carry = (zero,) * (3 * _UNROLL)
    inflight = issue(0, *bufs[0])
    for ci in range(_NCHUNK):
        cur = inflight
        if ci + 1 < _NCHUNK:
            inflight = issue(ci + 1, *bufs[(ci + 1) % 2])
        for cp in cur:
            cp.wait()
        pb, sb, xb, yb, zb, _ = bufs[ci % 2]
        carry = compute_chunk(pb, sb, xb, yb, zb, carry)

    cnt = carry[0] + carry[3]
    l1 = carry[1] + carry[4]
    al = carry[2] + carry[5]
    accb[pl.ds(0, _L)] = cnt
    accb[pl.ds(_L, _L)] = l1
    accb[pl.ds(2 * _L, _L)] = al
    pltpu.sync_copy(accb, out_hbm.at[wid])


def _sky_tc_body(pred_ref, sem_ref, nrm_ref, out_ref):
    first = (pl.program_id(0) == 0) & (pl.program_id(1) == 0)

    @pl.when(first)
    def _():
        out_ref[0] = 0.0
        out_ref[1] = 0.0
        out_ref[2] = 0.0

    sky = sem_ref[...] == _SKY_ID
    skyf = sky.astype(jnp.float32)
    p = pred_ref[...]
    x = nrm_ref[0, 0]
    y = nrm_ref[0, 1]
    z = nrm_ref[0, 2]
    nsq = jnp.maximum(x * x + y * y + z * z, 1e-16)
    e = y * lax.rsqrt(nsq)
    validf = skyf[0] * ((e > -0.999) & (e < 0.999)).astype(jnp.float32)
    out_ref[0] += jnp.sum(skyf)
    out_ref[1] += jnp.sum(jnp.abs(p - _REGRESS_VALUE) * skyf)
    out_ref[2] += jnp.sum((1.0 + e) * validf)


_sky_tc = pl.pallas_call(
    _sky_tc_body,
    grid=(_B - _B_SC, _H // _R_TC),
    in_specs=[
        pl.BlockSpec((1, _R_TC, _W), lambda b, i: (b + _B_SC, i, 0)),
        pl.BlockSpec((1, _R_TC, _W), lambda b, i: (b + _B_SC, i, 0)),
        pl.BlockSpec((1, 3, _R_TC, _W), lambda b, i: (b + _B_SC, 0, i, 0)),
    ],
    out_specs=pl.BlockSpec(memory_space=pltpu.SMEM),
    out_shape=jax.ShapeDtypeStruct((3,), jnp.float32),
    compiler_params=pltpu.CompilerParams(
        dimension_semantics=("arbitrary", "arbitrary")),
)


def kernel(prediction, target, prediction_normal, mask, sem_mask):
    del target, mask
    sem = sem_mask.astype(jnp.int32)
    sc_parts = _sky_sc(prediction, sem, prediction_normal)
    tc_parts = _sky_tc(prediction, sem, prediction_normal)
    sc_sums = sc_parts.reshape(_NW, 3, _L).sum(axis=(0, 2))
    cnt = sc_sums[0] + tc_parts[0]
    l1 = sc_sums[1] + tc_parts[1]
    al = sc_sums[2] + tc_parts[2]
    loss = (l1 + al) / (cnt + _EPS)
    loss = jnp.where(cnt > 0, loss, jnp.float32(0.0))
    loss = jnp.where(jnp.isnan(loss) | jnp.isinf(loss), jnp.float32(0.0), loss)
    return loss * _LOSS_WEIGHT


# speedup vs baseline: 1.1278x; 1.1278x over previous
import functools

import jax
import jax.numpy as jnp
from jax import lax
from jax.experimental import pallas as pl
from jax.experimental.pallas import tpu as pltpu
from jax.experimental.pallas import tpu_sc as plsc

_SKY_ID = 142
_LOSS_WEIGHT = 0.1
_REGRESS_VALUE = 1.8
_EPS = 1e-06

_B, _H, _W = 4, 512, 512

_NC, _NS, _L = 2, 16, 16
_NW = _NC * _NS
_WPB = _NW // _B
_SC_ROWS = 8
_R0_SC = _H - _WPB * _SC_ROWS
_UNROLL = 2
_GCOLS = _UNROLL * _L

_R_TC = 224

_mesh = plsc.VectorSubcoreMesh(core_axis_name="c", subcore_axis_name="s")


def _rsqrt(nsq):
    seed = jnp.int32(0x5F3759DF) - (lax.bitcast_convert_type(nsq, jnp.int32) >> 1)
    r = lax.bitcast_convert_type(seed, jnp.float32)
    h = 0.5 * nsq
    r = r * (1.5 - h * r * r)
    r = r * (1.5 - h * r * r)
    return r


@functools.partial(
    pl.kernel,
    mesh=_mesh,
    out_type=jax.ShapeDtypeStruct((_NW, 3 * _L), jnp.float32),
    scratch_types=[
        pltpu.VMEM((_SC_ROWS, _W), jnp.float32),
        pltpu.VMEM((_SC_ROWS, _W), jnp.int32),
        pltpu.VMEM((_SC_ROWS, _W), jnp.float32),
        pltpu.VMEM((_SC_ROWS, _W), jnp.float32),
        pltpu.VMEM((_SC_ROWS, _W), jnp.float32),
        pltpu.VMEM((3 * _L,), jnp.float32),
        pltpu.SemaphoreType.DMA,
    ],
)
def _sky_sc(pred_hbm, sem_hbm, nrm_hbm, out_hbm,
            pb, sb, xb, yb, zb, accb, dsem):
    wid = lax.axis_index("c") * _NS + lax.axis_index("s")
    b = wid >> 3
    rr = _R0_SC + (wid & 7) * _SC_ROWS
    prow = b * _H + rr
    nrow = (3 * b) * _H + rr

    cps = [
        pltpu.async_copy(pred_hbm.at[pl.ds(prow, _SC_ROWS), :], pb, dsem),
        pltpu.async_copy(sem_hbm.at[pl.ds(prow, _SC_ROWS), :], sb, dsem),
        pltpu.async_copy(nrm_hbm.at[pl.ds(nrow, _SC_ROWS), :], xb, dsem),
        pltpu.async_copy(nrm_hbm.at[pl.ds(nrow + _H, _SC_ROWS), :], yb, dsem),
        pltpu.async_copy(nrm_hbm.at[pl.ds(nrow + 2 * _H, _SC_ROWS), :], zb, dsem),
    ]
    for cp in cps:
        cp.wait()

    def body(g, carry):
        r = g >> 4
        c0 = (g & 15) * _GCOLS
        new = list(carry)
        for u in range(_UNROLL):
            cols = pl.ds(c0 + u * _L, _L)
            cnt, l1, al = new[3 * u], new[3 * u + 1], new[3 * u + 2]
            sky = sb[r, cols] == _SKY_ID
            p = pb[r, cols]
            x = xb[r, cols]
            y = yb[r, cols]
            z = zb[r, cols]
            cnt = cnt + jnp.where(sky, 1.0, 0.0).astype(jnp.float32)
            l1 = l1 + jnp.where(sky, jnp.abs(p - _REGRESS_VALUE), 0.0)
            nsq = jnp.maximum(x * x + y * y + z * z, 1e-16)
            e = y * _rsqrt(nsq)
            valid = sky & (e > -0.999) & (e < 0.999)
            al = al + jnp.where(valid, 1.0 + e, 0.0)
            new[3 * u], new[3 * u + 1], new[3 * u + 2] = cnt, l1, al
        return tuple(new)

    zero = jnp.zeros((_L,), jnp.float32)
    ngroups = _SC_ROWS * (_W // _GCOLS)
    carry = lax.fori_loop(0, ngroups, body, (zero,) * (3 * _UNROLL))

    accb[pl.ds(0, _L)] = carry[0] + carry[3]
    accb[pl.ds(_L, _L)] = carry[1] + carry[4]
    accb[pl.ds(2 * _L, _L)] = carry[2] + carry[5]
    pltpu.sync_copy(accb, out_hbm.at[wid])


def _sky_tc_body(pred_ref, sem_ref, nrm_ref, out_ref):
    first = (pl.program_id(0) == 0) & (pl.program_id(1) == 0)

    @pl.when(first)
    def _():
        out_ref[0] = 0.0
        out_ref[1] = 0.0
        out_ref[2] = 0.0

    sky = sem_ref[...] == _SKY_ID
    skyf = sky.astype(jnp.float32)
    p = pred_ref[...]
    x = nrm_ref[0, 0]
    y = nrm_ref[0, 1]
    z = nrm_ref[0, 2]
    nsq = jnp.maximum(x * x + y * y + z * z, 1e-16)
    e = y * lax.rsqrt(nsq)
    validf = skyf[0] * ((e > -0.999) & (e < 0.999)).astype(jnp.float32)
    out_ref[0] += jnp.sum(skyf)
    out_ref[1] += jnp.sum(jnp.abs(p - _REGRESS_VALUE) * skyf)
    out_ref[2] += jnp.sum((1.0 + e) * validf)


_sky_tc = pl.pallas_call(
    _sky_tc_body,
    grid=(_B, _R0_SC // _R_TC),
    in_specs=[
        pl.BlockSpec((1, _R_TC, _W), lambda b, i: (b, i, 0)),
        pl.BlockSpec((1, _R_TC, _W), lambda b, i: (b, i, 0)),
        pl.BlockSpec((1, 3, _R_TC, _W), lambda b, i: (b, 0, i, 0)),
    ],
    out_specs=pl.BlockSpec(memory_space=pltpu.SMEM),
    out_shape=jax.ShapeDtypeStruct((3,), jnp.float32),
    compiler_params=pltpu.CompilerParams(
        dimension_semantics=("arbitrary", "arbitrary")),
)


def kernel(prediction, target, prediction_normal, mask, sem_mask):
    del target, mask
    sem = sem_mask.astype(jnp.int32)
    pred2d = prediction.reshape(_B * _H, _W)
    sem2d = sem.reshape(_B * _H, _W)
    nrm2d = prediction_normal.reshape(_B * 3 * _H, _W)
    sc_parts = _sky_sc(pred2d, sem2d, nrm2d)
    tc_parts = _sky_tc(prediction, sem, prediction_normal)
    sc_sums = sc_parts.reshape(_NW, 3, _L).sum(axis=(0, 2))
    cnt = sc_sums[0] + tc_parts[0]
    l1 = sc_sums[1] + tc_parts[1]
    al = sc_sums[2] + tc_parts[2]
    loss = (l1 + al) / (cnt + _EPS)
    loss = jnp.where(cnt > 0, loss, jnp.float32(0.0))
    loss = jnp.where(jnp.isnan(loss) | jnp.isinf(loss), jnp.float32(0.0), loss)
    return loss * _LOSS_WEIGHT
